# baseline (speedup 1.0000x reference)
import functools

import jax
import jax.numpy as jnp
from jax import lax
from jax.experimental import pallas as pl
from jax.experimental.pallas import tpu as pltpu
from jax.experimental.pallas import tpu_sc as plsc

B, S, D, E, M, TOP_K = 2, 2048, 1024, 8, 2048, 2
T = B * S
A = TOP_K * T
EP = 128

TB = 512
BLK = 1024
R = A + E * BLK
NB = R // BLK

NSUB = 32
CHUNK = A // NSUB
LANES = 16


def _router_body(x_ref, g_ref, e1_ref, e2_ref, w1_ref, w2_ref,
                 ha_ref, hb_ref):
    logits = jnp.dot(x_ref[...], g_ref[...], preferred_element_type=jnp.float32)
    lane = lax.broadcasted_iota(jnp.int32, (TB, EP), 1)
    valid = lane < E
    logits = jnp.where(valid, logits, -jnp.inf)
    m = jnp.max(logits, axis=1, keepdims=True)
    ex = jnp.where(valid, jnp.exp(logits - m), 0.0)
    probs = ex / jnp.sum(ex, axis=1, keepdims=True)
    v1 = jnp.max(probs, axis=1, keepdims=True)
    i1 = jnp.min(jnp.where(probs == v1, lane, EP), axis=1, keepdims=True)
    probs2 = jnp.where(lane == i1, 0.0, probs)
    v2 = jnp.max(probs2, axis=1, keepdims=True)
    i2 = jnp.min(jnp.where(probs2 == v2, lane, EP), axis=1, keepdims=True)
    s = v1 + v2 + 1e-9
    e1_ref[...] = jnp.broadcast_to(i1, (TB, EP))
    e2_ref[...] = jnp.broadcast_to(i2, (TB, EP))
    w1_ref[...] = jnp.broadcast_to(v1 / s, (TB, EP))
    w2_ref[...] = jnp.broadcast_to(v2 / s, (TB, EP))
    oh1 = (lane == i1).astype(jnp.int32)
    oh2 = (lane == i2).astype(jnp.int32)
    ha_ref[...] = jnp.concatenate(
        [jnp.sum(oh1[:CHUNK], axis=0, keepdims=True),
         jnp.sum(oh1[CHUNK:], axis=0, keepdims=True)], axis=0)[None]
    hb_ref[...] = jnp.concatenate(
        [jnp.sum(oh2[:CHUNK], axis=0, keepdims=True),
         jnp.sum(oh2[CHUNK:], axis=0, keepdims=True)], axis=0)[None]


def _router(t2d, gate_pad):
    return pl.pallas_call(
        _router_body,
        grid=(T // TB,),
        in_specs=[
            pl.BlockSpec((TB, D), lambda i: (i, 0)),
            pl.BlockSpec((D, EP), lambda i: (0, 0)),
        ],
        out_specs=[
            pl.BlockSpec((TB, EP), lambda i: (i, 0)),
            pl.BlockSpec((TB, EP), lambda i: (i, 0)),
            pl.BlockSpec((TB, EP), lambda i: (i, 0)),
            pl.BlockSpec((TB, EP), lambda i: (i, 0)),
            pl.BlockSpec((1, 2, EP), lambda i: (i, 0, 0)),
            pl.BlockSpec((1, 2, EP), lambda i: (i, 0, 0)),
        ],
        out_shape=[
            jax.ShapeDtypeStruct((T, EP), jnp.int32),
            jax.ShapeDtypeStruct((T, EP), jnp.int32),
            jax.ShapeDtypeStruct((T, EP), jnp.float32),
            jax.ShapeDtypeStruct((T, EP), jnp.float32),
            jax.ShapeDtypeStruct((T // TB, 2, EP), jnp.int32),
            jax.ShapeDtypeStruct((T // TB, 2, EP), jnp.int32),
        ],
    )(t2d, gate_pad)


def _sc_mesh():
    return plsc.VectorSubcoreMesh(core_axis_name="c", subcore_axis_name="s")


_SC_PARAMS = pltpu.CompilerParams(needs_layout_passes=False)


def _wid():
    return lax.axis_index("s") * 2 + lax.axis_index("c")


def _iconst(val):
    return jnp.full((LANES,), val, jnp.int32)


_GDN = lax.GatherDimensionNumbers(
    offset_dims=(), collapsed_slice_dims=(0,), start_index_map=(0,))


def _splat_lane(vec, lane_idx):
    return lax.gather(vec, _iconst(lane_idx)[:, None], _GDN, slice_sizes=(1,),
                      mode=lax.GatherScatterMode.PROMISE_IN_BOUNDS)


def _dispatch_body(eid_hbm, x_hbm, wcat_hbm, hista_hbm, histb_hbm,
                   xs_hbm, ws_hbm, pos_hbm,
                   eid_v, hista_v, histb_v, tok_v, dest_v, xr_v, wr_v,
                   semg, sems, semw):
    wid = _wid()
    base = wid * CHUNK
    widv = jnp.full((LANES,), wid, jnp.int32)
    tokbase = base - jnp.where(wid >= NSUB // TOP_K, T, 0)
    tokbasev = jnp.full((LANES,), tokbase, jnp.int32)
    pltpu.sync_copy(eid_hbm.at[pl.ds(base, CHUNK)], eid_v)
    pltpu.sync_copy(hista_hbm, hista_v)
    pltpu.sync_copy(histb_hbm, histb_v)

    lane = lax.iota(jnp.int32, LANES)
    zero = _iconst(0)
    one = _iconst(1)
    tot = zero
    before = zero
    for r in range(NSUB):
        if r < NSUB // 2:
            row = hista_v[r, pl.ds(0, LANES)]
        else:
            row = histb_v[r - NSUB // 2, pl.ds(0, LANES)]
        tot = tot + row
        before = before + jnp.where(_iconst(r) < widv, row, zero)
    padded = ((tot + _iconst(BLK - 1)) // _iconst(BLK)) * _iconst(BLK)
    starts = plsc.cumsum(padded) - padded
    off = starts + before

    for c in range(CHUNK // LANES):
        v = eid_v[pl.ds(c * LANES, LANES)]
        toks = tokbasev + _iconst(c * LANES) + lane
        dest = zero
        for e in range(E):
            m = v == _iconst(e)
            mi = jnp.where(m, one, zero)
            rnk = plsc.cumsum(mi)
            off_e = _splat_lane(off, e)
            dest = jnp.where(m, off_e + rnk - one, dest)
            cnt = plsc.all_reduce_population_count(m)
            off = off + jnp.where(lane == _iconst(e), cnt, zero)
        j, k = c // 2, c % 2
        tok_v[j, pl.ds(k * LANES, LANES)] = toks
        dest_v[j, pl.ds(k * LANES, LANES)] = dest

    def fire(j):
        b = j % 2
        cw = pltpu.async_copy(
            wcat_hbm.at[pl.ds(base + j * _NRW, _NRW)], wr_v.at[b], semw)
        cg = pltpu.async_copy(x_hbm.at[tok_v.at[j]], xr_v.at[b], semg)
        return cg, cw

    pend = fire(0)
    scat = None
    for j in range(_NRD):
        b = j % 2
        pend[0].wait()
        pend[1].wait()
        if j + 1 < _NRD:
            if scat is not None:
                scat[0].wait()
                scat[1].wait()
                scat = None
            pend = fire(j + 1)
        s1 = pltpu.async_copy(xr_v.at[b], xs_hbm.at[dest_v.at[j]], sems)
        s2 = pltpu.async_copy(wr_v.at[b], ws_hbm.at[dest_v.at[j]], sems)
        pltpu.sync_copy(dest_v.at[j], pos_hbm.at[pl.ds(base + j * _NRW, _NRW)])
        scat = (s1, s2)
    scat[0].wait()
    scat[1].wait()


_NRW = 32
_NRD = CHUNK // _NRW

_dispatch_kernel = functools.partial(
    pl.kernel,
    out_type=[
        jax.ShapeDtypeStruct((R, D), jnp.float32),
        jax.ShapeDtypeStruct((R, EP), jnp.float32),
        jax.ShapeDtypeStruct((A,), jnp.int32),
    ],
    mesh=_sc_mesh(),
    scratch_types=[
        pltpu.VMEM((CHUNK,), jnp.int32),
        pltpu.VMEM((NSUB // 2, EP), jnp.int32),
        pltpu.VMEM((NSUB // 2, EP), jnp.int32),
        pltpu.VMEM((_NRD, _NRW), jnp.int32),
        pltpu.VMEM((_NRD, _NRW), jnp.int32),
        pltpu.VMEM((2, _NRW, D), jnp.float32),
        pltpu.VMEM((2, _NRW, EP), jnp.float32),
        pltpu.SemaphoreType.DMA,
        pltpu.SemaphoreType.DMA,
        pltpu.SemaphoreType.DMA,
    ],
    compiler_params=_SC_PARAMS,
)(_dispatch_body)


def _moe_body(be_ref, xs_ref, wi_ref, wo_ref, ws_ref, os_ref):
    xb = xs_ref[...].astype(jnp.bfloat16)
    h = jnp.dot(xb, wi_ref[0], preferred_element_type=jnp.float32)
    h = jax.nn.gelu(h)
    o = jnp.dot(h.astype(jnp.bfloat16), wo_ref[0],
                preferred_element_type=jnp.float32)
    os_ref[...] = o * ws_ref[:, 0:1]


def _grouped_moe(block_expert, xs, wi, wo, ws):
    grid_spec = pltpu.PrefetchScalarGridSpec(
        num_scalar_prefetch=1,
        grid=(NB,),
        in_specs=[
            pl.BlockSpec((BLK, D), lambda i, be: (i, 0)),
            pl.BlockSpec((1, D, M), lambda i, be: (be[i], 0, 0)),
            pl.BlockSpec((1, M, D), lambda i, be: (be[i], 0, 0)),
            pl.BlockSpec((BLK, EP), lambda i, be: (i, 0)),
        ],
        out_specs=pl.BlockSpec((BLK, D), lambda i, be: (i, 0)),
    )
    return pl.pallas_call(
        _moe_body,
        grid_spec=grid_spec,
        out_shape=jax.ShapeDtypeStruct((R, D), jnp.float32),
        compiler_params=pltpu.CompilerParams(
            dimension_semantics=("arbitrary",),
        ),
    )(block_expert, xs, wi, wo, ws)


def _combine_body(os_hbm, pos_hbm, out_hbm, p0_v, p1_v, o_v, r1_v,
                  sem0, sem1):
    wid = _wid()
    tper = T // NSUB
    ngrp = tper // LANES

    def fire(g):
        b = g % 2
        t0 = wid * tper + g * LANES
        pltpu.sync_copy(pos_hbm.at[pl.ds(t0, LANES)], p0_v.at[b])
        pltpu.sync_copy(pos_hbm.at[pl.ds(T + t0, LANES)], p1_v.at[b])
        c0 = pltpu.async_copy(os_hbm.at[p0_v.at[b]], o_v.at[b], sem0)
        c1 = pltpu.async_copy(os_hbm.at[p1_v.at[b]], r1_v.at[b], sem1)
        return c0, c1

    pend = fire(0)
    for g in range(ngrp):
        b = g % 2
        pend[0].wait()
        pend[1].wait()
        if g + 1 < ngrp:
            pend = fire(g + 1)

        def body(i, _):
            for j in range(D // LANES):
                sl = pl.ds(j * LANES, LANES)
                plsc.addupdate(o_v.at[b, i, sl], r1_v[b, i, sl])
            return 0

        lax.fori_loop(0, LANES, body, 0)
        t0 = wid * tper + g * LANES
        pltpu.sync_copy(o_v.at[b], out_hbm.at[pl.ds(t0, LANES)])


_combine_kernel = functools.partial(
    pl.kernel,
    out_type=jax.ShapeDtypeStruct((T, D), jnp.float32),
    mesh=_sc_mesh(),
    scratch_types=[
        pltpu.VMEM((2, LANES), jnp.int32),
        pltpu.VMEM((2, LANES), jnp.int32),
        pltpu.VMEM((2, LANES, D), jnp.float32),
        pltpu.VMEM((2, LANES, D), jnp.float32),
        pltpu.SemaphoreType.DMA,
        pltpu.SemaphoreType.DMA,
    ],
    compiler_params=_SC_PARAMS,
)(_combine_body)


def kernel(x, gate_kernel, wi, wo):
    t2d = x.reshape(T, D)
    gate_pad = jnp.zeros((D, EP), jnp.float32).at[:, :E].set(gate_kernel)

    e1b, e2b, w1b, w2b, ha, hb = _router(t2d, gate_pad)
    eid = jnp.concatenate([e1b[:, 0], e2b[:, 0]])
    wcat = jnp.concatenate([w1b, w2b], axis=0)
    hista = ha.reshape(T // CHUNK, EP)
    histb = hb.reshape(T // CHUNK, EP)

    xs, ws, pos = _dispatch_kernel(eid, t2d, wcat, hista, histb)

    cnt = jnp.sum(hista[:, :E], axis=0) + jnp.sum(histb[:, :E], axis=0)
    pad_blocks = (cnt + (BLK - 1)) // BLK
    ends = jnp.cumsum(pad_blocks)
    bi = jnp.arange(NB, dtype=ends.dtype)
    block_expert = jnp.minimum(
        jnp.searchsorted(ends, bi, side="right"), E - 1
    ).astype(jnp.int32)

    os = _grouped_moe(block_expert, xs,
                      wi.astype(jnp.bfloat16), wo.astype(jnp.bfloat16), ws)
    out = _combine_kernel(os, pos)
    return out.reshape(B, S, D)


# speedup vs baseline: 1.1127x; 1.1127x over previous
import functools

import jax
import jax.numpy as jnp
from jax import lax
from jax.experimental import pallas as pl
from jax.experimental.pallas import tpu as pltpu
from jax.experimental.pallas import tpu_sc as plsc

B, S, D, E, M, TOP_K = 2, 2048, 1024, 8, 2048, 2
T = B * S
A = TOP_K * T
EP = 128

TB = 512
BLK = 512
R = A + E * BLK
NB = R // BLK

NSUB = 32
CHUNK = A // NSUB
LANES = 16


def _router_body(x_ref, g_ref, e1_ref, e2_ref, w1_ref, w2_ref,
                 ha_ref, hb_ref):
    logits = jnp.dot(x_ref[...], g_ref[...], preferred_element_type=jnp.float32)
    lane = lax.broadcasted_iota(jnp.int32, (TB, EP), 1)
    valid = lane < E
    logits = jnp.where(valid, logits, -jnp.inf)
    m = jnp.max(logits, axis=1, keepdims=True)
    ex = jnp.where(valid, jnp.exp(logits - m), 0.0)
    probs = ex / jnp.sum(ex, axis=1, keepdims=True)
    v1 = jnp.max(probs, axis=1, keepdims=True)
    i1 = jnp.min(jnp.where(probs == v1, lane, EP), axis=1, keepdims=True)
    probs2 = jnp.where(lane == i1, 0.0, probs)
    v2 = jnp.max(probs2, axis=1, keepdims=True)
    i2 = jnp.min(jnp.where(probs2 == v2, lane, EP), axis=1, keepdims=True)
    s = v1 + v2 + 1e-9
    e1_ref[...] = jnp.broadcast_to(i1, (TB, EP))
    e2_ref[...] = jnp.broadcast_to(i2, (TB, EP))
    w1_ref[...] = jnp.broadcast_to(v1 / s, (TB, EP))
    w2_ref[...] = jnp.broadcast_to(v2 / s, (TB, EP))
    oh1 = (lane == i1).astype(jnp.int32)
    oh2 = (lane == i2).astype(jnp.int32)
    ha_ref[...] = jnp.concatenate(
        [jnp.sum(oh1[:CHUNK], axis=0, keepdims=True),
         jnp.sum(oh1[CHUNK:], axis=0, keepdims=True)], axis=0)[None]
    hb_ref[...] = jnp.concatenate(
        [jnp.sum(oh2[:CHUNK], axis=0, keepdims=True),
         jnp.sum(oh2[CHUNK:], axis=0, keepdims=True)], axis=0)[None]


def _router(t2d, gate_pad):
    return pl.pallas_call(
        _router_body,
        grid=(T // TB,),
        in_specs=[
            pl.BlockSpec((TB, D), lambda i: (i, 0)),
            pl.BlockSpec((D, EP), lambda i: (0, 0)),
        ],
        out_specs=[
            pl.BlockSpec((TB, EP), lambda i: (i, 0)),
            pl.BlockSpec((TB, EP), lambda i: (i, 0)),
            pl.BlockSpec((TB, EP), lambda i: (i, 0)),
            pl.BlockSpec((TB, EP), lambda i: (i, 0)),
            pl.BlockSpec((1, 2, EP), lambda i: (i, 0, 0)),
            pl.BlockSpec((1, 2, EP), lambda i: (i, 0, 0)),
        ],
        out_shape=[
            jax.ShapeDtypeStruct((T, EP), jnp.int32),
            jax.ShapeDtypeStruct((T, EP), jnp.int32),
            jax.ShapeDtypeStruct((T, EP), jnp.float32),
            jax.ShapeDtypeStruct((T, EP), jnp.float32),
            jax.ShapeDtypeStruct((T // TB, 2, EP), jnp.int32),
            jax.ShapeDtypeStruct((T // TB, 2, EP), jnp.int32),
        ],
    )(t2d, gate_pad)


def _sc_mesh():
    return plsc.VectorSubcoreMesh(core_axis_name="c", subcore_axis_name="s")


_SC_PARAMS = pltpu.CompilerParams(needs_layout_passes=False)


def _wid():
    return lax.axis_index("s") * 2 + lax.axis_index("c")


def _iconst(val):
    return jnp.full((LANES,), val, jnp.int32)


_GDN = lax.GatherDimensionNumbers(
    offset_dims=(), collapsed_slice_dims=(0,), start_index_map=(0,))


def _splat_lane(vec, lane_idx):
    return lax.gather(vec, _iconst(lane_idx)[:, None], _GDN, slice_sizes=(1,),
                      mode=lax.GatherScatterMode.PROMISE_IN_BOUNDS)


def _dispatch_body(eid_hbm, x_hbm, wcat_hbm, hista_hbm, histb_hbm,
                   xs_hbm, ws_hbm, pos_hbm,
                   eid_v, hista_v, histb_v, tok_v, dest_v, xr_v, wr_v,
                   semg, sems, semw):
    wid = _wid()
    base = wid * CHUNK
    widv = jnp.full((LANES,), wid, jnp.int32)
    tokbase = base - jnp.where(wid >= NSUB // TOP_K, T, 0)
    tokbasev = jnp.full((LANES,), tokbase, jnp.int32)
    pltpu.sync_copy(eid_hbm.at[pl.ds(base, CHUNK)], eid_v)
    pltpu.sync_copy(hista_hbm, hista_v)
    pltpu.sync_copy(histb_hbm, histb_v)

    lane = lax.iota(jnp.int32, LANES)
    zero = _iconst(0)
    one = _iconst(1)
    tot = zero
    before = zero
    for r in range(NSUB):
        if r < NSUB // 2:
            row = hista_v[r, pl.ds(0, LANES)]
        else:
            row = histb_v[r - NSUB // 2, pl.ds(0, LANES)]
        tot = tot + row
        before = before + jnp.where(_iconst(r) < widv, row, zero)
    padded = ((tot + _iconst(BLK - 1)) // _iconst(BLK)) * _iconst(BLK)
    starts = plsc.cumsum(padded) - padded
    off = starts + before

    for c in range(CHUNK // LANES):
        v = eid_v[pl.ds(c * LANES, LANES)]
        toks = tokbasev + _iconst(c * LANES) + lane
        dest = zero
        for e in range(E):
            m = v == _iconst(e)
            mi = jnp.where(m, one, zero)
            rnk = plsc.cumsum(mi)
            off_e = _splat_lane(off, e)
            dest = jnp.where(m, off_e + rnk - one, dest)
            cnt = plsc.all_reduce_population_count(m)
            off = off + jnp.where(lane == _iconst(e), cnt, zero)
        j, k = c // 2, c % 2
        tok_v[j, pl.ds(k * LANES, LANES)] = toks
        dest_v[j, pl.ds(k * LANES, LANES)] = dest

    def fire(j):
        b = j % 2
        cw = pltpu.async_copy(
            wcat_hbm.at[pl.ds(base + j * _NRW, _NRW)], wr_v.at[b], semw)
        cg = pltpu.async_copy(x_hbm.at[tok_v.at[j]], xr_v.at[b], semg)
        return cg, cw

    pend = fire(0)
    scat = None
    for j in range(_NRD):
        b = j % 2
        pend[0].wait()
        pend[1].wait()
        if j + 1 < _NRD:
            if scat is not None:
                scat[0].wait()
                scat[1].wait()
                scat = None
            pend = fire(j + 1)
        s1 = pltpu.async_copy(xr_v.at[b], xs_hbm.at[dest_v.at[j]], sems)
        s2 = pltpu.async_copy(wr_v.at[b], ws_hbm.at[dest_v.at[j]], sems)
        pltpu.sync_copy(dest_v.at[j], pos_hbm.at[pl.ds(base + j * _NRW, _NRW)])
        scat = (s1, s2)
    scat[0].wait()
    scat[1].wait()


_NRW = 32
_NRD = CHUNK // _NRW

_dispatch_kernel = functools.partial(
    pl.kernel,
    out_type=[
        jax.ShapeDtypeStruct((R, D), jnp.float32),
        jax.ShapeDtypeStruct((R, EP), jnp.float32),
        jax.ShapeDtypeStruct((A,), jnp.int32),
    ],
    mesh=_sc_mesh(),
    scratch_types=[
        pltpu.VMEM((CHUNK,), jnp.int32),
        pltpu.VMEM((NSUB // 2, EP), jnp.int32),
        pltpu.VMEM((NSUB // 2, EP), jnp.int32),
        pltpu.VMEM((_NRD, _NRW), jnp.int32),
        pltpu.VMEM((_NRD, _NRW), jnp.int32),
        pltpu.VMEM((2, _NRW, D), jnp.float32),
        pltpu.VMEM((2, _NRW, EP), jnp.float32),
        pltpu.SemaphoreType.DMA,
        pltpu.SemaphoreType.DMA,
        pltpu.SemaphoreType.DMA,
    ],
    compiler_params=_SC_PARAMS,
)(_dispatch_body)


def _moe_body(be_ref, xs_ref, wi_ref, wo_ref, ws_ref, os_ref):
    xb = xs_ref[...].astype(jnp.bfloat16)
    h = jnp.dot(xb, wi_ref[0], preferred_element_type=jnp.float32)
    h = jax.nn.gelu(h)
    o = jnp.dot(h.astype(jnp.bfloat16), wo_ref[0],
                preferred_element_type=jnp.float32)
    os_ref[...] = o * ws_ref[:, 0:1]


def _grouped_moe(block_expert, xs, wi, wo, ws):
    grid_spec = pltpu.PrefetchScalarGridSpec(
        num_scalar_prefetch=1,
        grid=(NB,),
        in_specs=[
            pl.BlockSpec((BLK, D), lambda i, be: (i, 0)),
            pl.BlockSpec((1, D, M), lambda i, be: (be[i], 0, 0)),
            pl.BlockSpec((1, M, D), lambda i, be: (be[i], 0, 0)),
            pl.BlockSpec((BLK, EP), lambda i, be: (i, 0)),
        ],
        out_specs=pl.BlockSpec((BLK, D), lambda i, be: (i, 0)),
    )
    return pl.pallas_call(
        _moe_body,
        grid_spec=grid_spec,
        out_shape=jax.ShapeDtypeStruct((R, D), jnp.float32),
        compiler_params=pltpu.CompilerParams(
            dimension_semantics=("arbitrary",),
        ),
    )(block_expert, xs, wi, wo, ws)


def _combine_body(os_hbm, pos_hbm, out_hbm, p0_v, p1_v, o_v, r1_v,
                  sem0, sem1):
    wid = _wid()
    tper = T // NSUB
    ngrp = tper // LANES

    def fire(g):
        b = g % 2
        t0 = wid * tper + g * LANES
        pltpu.sync_copy(pos_hbm.at[pl.ds(t0, LANES)], p0_v.at[b])
        pltpu.sync_copy(pos_hbm.at[pl.ds(T + t0, LANES)], p1_v.at[b])
        c0 = pltpu.async_copy(os_hbm.at[p0_v.at[b]], o_v.at[b], sem0)
        c1 = pltpu.async_copy(os_hbm.at[p1_v.at[b]], r1_v.at[b], sem1)
        return c0, c1

    pend = fire(0)
    for g in range(ngrp):
        b = g % 2
        pend[0].wait()
        pend[1].wait()
        if g + 1 < ngrp:
            pend = fire(g + 1)

        def body(i, _):
            for j in range(D // LANES):
                sl = pl.ds(j * LANES, LANES)
                plsc.addupdate(o_v.at[b, i, sl], r1_v[b, i, sl])
            return 0

        lax.fori_loop(0, LANES, body, 0)
        t0 = wid * tper + g * LANES
        pltpu.sync_copy(o_v.at[b], out_hbm.at[pl.ds(t0, LANES)])


_combine_kernel = functools.partial(
    pl.kernel,
    out_type=jax.ShapeDtypeStruct((T, D), jnp.float32),
    mesh=_sc_mesh(),
    scratch_types=[
        pltpu.VMEM((2, LANES), jnp.int32),
        pltpu.VMEM((2, LANES), jnp.int32),
        pltpu.VMEM((2, LANES, D), jnp.float32),
        pltpu.VMEM((2, LANES, D), jnp.float32),
        pltpu.SemaphoreType.DMA,
        pltpu.SemaphoreType.DMA,
    ],
    compiler_params=_SC_PARAMS,
)(_combine_body)


def kernel(x, gate_kernel, wi, wo):
    t2d = x.reshape(T, D)
    gate_pad = jnp.zeros((D, EP), jnp.float32).at[:, :E].set(gate_kernel)

    e1b, e2b, w1b, w2b, ha, hb = _router(t2d, gate_pad)
    eid = jnp.concatenate([e1b[:, 0], e2b[:, 0]])
    wcat = jnp.concatenate([w1b, w2b], axis=0)
    hista = ha.reshape(T // CHUNK, EP)
    histb = hb.reshape(T // CHUNK, EP)

    xs, ws, pos = _dispatch_kernel(eid, t2d, wcat, hista, histb)

    cnt = jnp.sum(hista[:, :E], axis=0) + jnp.sum(histb[:, :E], axis=0)
    pad_blocks = (cnt + (BLK - 1)) // BLK
    ends = jnp.cumsum(pad_blocks)
    bi = jnp.arange(NB, dtype=ends.dtype)
    block_expert = jnp.minimum(
        jnp.searchsorted(ends, bi, side="right"), E - 1
    ).astype(jnp.int32)

    os = _grouped_moe(block_expert, xs,
                      wi.astype(jnp.bfloat16), wo.astype(jnp.bfloat16), ws)
    out = _combine_kernel(os, pos)
    return out.reshape(B, S, D)
